# baseline (speedup 1.0000x reference)
import functools

import jax
import jax.numpy as jnp
import numpy as np
from jax import lax
from jax.experimental import pallas as pl
from jax.experimental.pallas import tpu as pltpu
from jax.experimental.pallas import tpu_sc as plsc

_NODES = [8, 32, 128, 512, 2048]
_BAND_BLK = 256
_BAND_HALO = 128
_BSZ = 32


def _pool_matrix(v):
    h = int(round((v / 2) ** 0.5))
    w = 2 * h
    p = np.zeros((v, v // 4), np.float32)
    for h2 in range(h // 2):
        for w2 in range(w // 2):
            col = h2 * (w // 2) + w2
            for dh in (0, 1):
                for dw in (0, 1):
                    p[(2 * h2 + dh) * w + (2 * w2 + dw), col] = 0.25
    return p


_POOL = {v: _pool_matrix(v) for v in _NODES[1:]}
_UNPOOL = {v: (4.0 * _POOL[v].T).copy() for v in _NODES[1:]}
_EYE = np.eye(_BSZ, dtype=np.float32)
_PK32 = np.kron(_EYE, _POOL[32])
_UK32 = np.kron(_EYE, _UNPOOL[32])


_SC_TECS = 32


def _sc_build_flat(dst, vals, pad_total, ch):
    tot = dst.shape[0]
    mesh = plsc.VectorSubcoreMesh(core_axis_name="c", subcore_axis_name="s")

    def body(dst_hbm, vals_hbm, out_hbm, dst_v, vals_v, chunk_v):
        wid = lax.axis_index("s") * 2 + lax.axis_index("c")
        lo = wid * ch
        pltpu.sync_copy(dst_hbm, dst_v)
        pltpu.sync_copy(vals_hbm, vals_v)
        zv = jnp.zeros((16,), jnp.float32)

        def zbody(i, carry):
            chunk_v[pl.ds(i * 16, 16)] = zv
            return carry

        lax.fori_loop(0, ch // 16, zbody, 0)

        def sbody(i, carry):
            d = dst_v[pl.ds(i * 16, 16)]
            v = vals_v[pl.ds(i * 16, 16)]
            dl = d - lo
            m = (d >= lo) & (dl < ch)
            plsc.store_scatter(chunk_v, [dl], v, mask=m)
            return carry

        lax.fori_loop(0, tot // 16, sbody, 0)
        pltpu.sync_copy(chunk_v, out_hbm.at[pl.ds(lo, ch)])

    return pl.kernel(
        body,
        out_type=jax.ShapeDtypeStruct((pad_total,), jnp.float32),
        mesh=mesh,
        compiler_params=pltpu.CompilerParams(needs_layout_passes=False),
        scratch_types=[
            pltpu.VMEM((tot,), jnp.int32),
            pltpu.VMEM((tot,), jnp.float32),
            pltpu.VMEM((ch,), jnp.float32),
        ],
    )(dst, vals)


def _dst_dense(lap, v, base):
    rows, cols, _ = lap
    return base + rows * v + cols


def _dst_band(lap, base):
    rows, cols, _ = lap
    j = cols // _BAND_BLK
    rloc = rows - j * _BAND_BLK + _BAND_HALO
    win = _BAND_BLK + 2 * _BAND_HALO
    return base + (j * win + rloc) * _BAND_BLK + cols % _BAND_BLK


def _kron_lift(d, bsz):
    v = d.shape[0]
    eye = jnp.asarray(np.eye(bsz, dtype=np.float32))
    return (eye[:, None, :, None] * d[None, :, None, :]).reshape(
        bsz * v, bsz * v)


def _dot(a, b):
    return jnp.dot(a, b, preferred_element_type=jnp.float32)


def _apply_l(z, l_ref, banded):
    if not banded:
        return _dot(z, l_ref[...])
    nblk = l_ref.shape[0]
    zp = jnp.pad(z, ((0, 0), (_BAND_HALO, _BAND_HALO)))
    win = _BAND_BLK + 2 * _BAND_HALO
    outs = []
    for j in range(nblk):
        outs.append(_dot(zp[:, j * _BAND_BLK:j * _BAND_BLK + win], l_ref[j]))
    return jnp.concatenate(outs, axis=1)


def _coarse_body(x_ref, *refs, relu, has_pm, has_skip, post, fo):
    i = 0
    s_ref = pm_ref = None
    if has_skip:
        s_ref = refs[i]; i += 1
    if has_pm:
        pm_ref = refs[i]; i += 1
    l_ref, w_ref, b_ref = refs[i:i + 3]
    i += 3
    ws_ref = None
    if has_skip and post:
        ws_ref = refs[i]; i += 1
    o_ref = refs[i]

    x = x_ref[...]
    if not post:
        if has_pm:
            x = _dot(x, pm_ref[...])
        if has_skip:
            x = jnp.concatenate([x, s_ref[...]], axis=0)
        x0 = x
        x1 = _dot(x0, l_ref[...])
        x2 = 2.0 * _dot(x1, l_ref[...]) - x0
        y = _dot(w_ref[...], jnp.concatenate([x0, x1, x2], axis=0))
    else:
        y3 = _dot(w_ref[...], x)
        if has_skip:
            y3 = y3 + _dot(ws_ref[...], s_ref[...])
        if has_pm:
            y3 = _dot(y3, pm_ref[...])
        y0, y1, y2 = y3[:fo], y3[fo:2 * fo], y3[2 * fo:]
        t = _dot(y2, l_ref[...])
        y = (y0 - y2) + _dot(y1 + 2.0 * t, l_ref[...])

    y = y + b_ref[...]
    if relu:
        y = jnp.maximum(y, 0.0)
    o_ref[...] = y


def _cheb_pre(x, l, w, b, relu):
    x1 = _dot(x, l)
    x2 = 2.0 * _dot(x1, l) - x
    y = _dot(w, jnp.concatenate([x, x1, x2], axis=0)) + b
    return jnp.maximum(y, 0.0) if relu else y


def _cheb_post(y3, l, b, fo, relu):
    y0, y1, y2 = y3[:fo], y3[fo:2 * fo], y3[2 * fo:]
    t = _dot(y2, l)
    y = (y0 - y2) + _dot(y1 + 2.0 * t, l) + b
    return jnp.maximum(y, 0.0) if relu else y


def _coarse_chain_body(x2f_ref, lk1_ref, lk2_ref, pk_ref, uk_ref,
                       w1_ref, b1_ref, w2_ref, b2_ref, w3_ref, b3_ref,
                       w4_ref, b4_ref, w5_ref, b5_ref,
                       w6h_ref, w6s_ref, b6_ref, o_ref):
    x2f = x2f_ref[...]
    lk1 = lk1_ref[...]
    lk2 = lk2_ref[...]
    x1f = _cheb_pre(_dot(x2f, pk_ref[...]), lk1, w1_ref[...], b1_ref[...],
                    True)
    x0f = _cheb_pre(x1f, lk1, w2_ref[...], b2_ref[...], False)
    h = _cheb_pre(x0f, lk1, w3_ref[...], b3_ref[...], True)
    h = _cheb_pre(jnp.concatenate([h, x1f], axis=0), lk1, w4_ref[...],
                  b4_ref[...], True)
    y3 = _dot(_dot(w5_ref[...], h), uk_ref[...])
    fo = b5_ref.shape[0]
    h = _cheb_post(y3, lk2, b5_ref[...], fo, True)
    y3 = _dot(w6h_ref[...], h) + _dot(w6s_ref[...], x2f)
    o_ref[...] = _cheb_post(y3, lk2, b6_ref[...], fo, True)


def _enc5_pair_body(x_ref, l_ref, w1_ref, b1_ref, w2_ref, b2_ref, o_ref, *,
                    nb):
    def cheb(x3, w, b):
        f, v = x3.shape[1], x3.shape[2]
        xm = x3.reshape(nb * f, v)
        x1 = _apply_l(xm, l_ref, True)
        x2 = 2.0 * _apply_l(x1, l_ref, True) - xm
        ys = []
        for bi in range(nb):
            s = slice(bi * f, (bi + 1) * f)
            ys.append(_dot(w, jnp.concatenate([xm[s], x1[s], x2[s]], axis=0)))
        return jnp.stack(ys) + b

    y = jnp.maximum(cheb(x_ref[...], w1_ref[...], b1_ref[...]), 0.0)
    o_ref[...] = jnp.maximum(cheb(y, w2_ref[...], b2_ref[...]), 0.0)


def _dec5_pair_body(x_ref, l_ref, pm_ref, w1_ref, b1_ref, w2_ref, b2_ref,
                    o_ref, *, nb):
    def cheb_post(x3, w, b, pm):
        fo3 = w.shape[0]
        fo = fo3 // 3
        y3m = jnp.concatenate([_dot(w, x3[bi]) for bi in range(nb)], axis=0)
        if pm is not None:
            y3m = _dot(y3m, pm)
        v = y3m.shape[1]
        y3d = y3m.reshape(nb, fo3, v)
        y0 = y3d[:, :fo, :].reshape(nb * fo, v)
        y1 = y3d[:, fo:2 * fo, :].reshape(nb * fo, v)
        y2 = y3d[:, 2 * fo:, :].reshape(nb * fo, v)
        t = _apply_l(y2, l_ref, True)
        y = (y0 - y2) + _apply_l(y1 + 2.0 * t, l_ref, True)
        return y.reshape(nb, fo, v) + b

    y = cheb_post(x_ref[...], w1_ref[...], b1_ref[...], pm_ref[...])
    o_ref[...] = cheb_post(y, w2_ref[...], b2_ref[...], None)


def _dec_pair_body(x_ref, l_ref, pm_ref, s_ref, w1_ref, b1_ref, w2h_ref,
                   w2s_ref, b2_ref, o_ref, *, nb):
    def cheb_post(x3, w, b, pm, skip3, ws):
        fo3 = w.shape[0]
        fo = fo3 // 3
        parts = []
        for bi in range(nb):
            yb = _dot(w, x3[bi])
            if skip3 is not None:
                yb = yb + _dot(ws, skip3[bi])
            parts.append(yb)
        y3m = jnp.concatenate(parts, axis=0)
        if pm is not None:
            y3m = _dot(y3m, pm)
        v = y3m.shape[1]
        y3d = y3m.reshape(nb, fo3, v)
        y0 = y3d[:, :fo, :].reshape(nb * fo, v)
        y1 = y3d[:, fo:2 * fo, :].reshape(nb * fo, v)
        y2 = y3d[:, 2 * fo:, :].reshape(nb * fo, v)
        t = _dot(y2, l_ref[...])
        y = (y0 - y2) + _dot(y1 + 2.0 * t, l_ref[...])
        return jnp.maximum(y.reshape(nb, fo, v) + b, 0.0)

    y = cheb_post(x_ref[...], w1_ref[...], b1_ref[...], pm_ref[...],
                  None, None)
    o_ref[...] = cheb_post(y, w2h_ref[...], b2_ref[...], None,
                           s_ref[...], w2s_ref[...])


def _enc32_body(x_ref, l3_ref, l2_ref, p5_ref, p1_ref, w3_ref, b3_ref,
                w2_ref, b2_ref, o3_ref, o2_ref, *, nb):
    def block(x3, pm, l, w, b):
        f, v = x3.shape[1], x3.shape[2]
        xm = _dot(x3.reshape(nb * f, v), pm)
        vout = xm.shape[1]
        x1 = _dot(xm, l)
        x2 = 2.0 * _dot(x1, l) - xm
        ys = []
        for bi in range(nb):
            s = slice(bi * f, (bi + 1) * f)
            ys.append(_dot(w, jnp.concatenate([xm[s], x1[s], x2[s]],
                                              axis=0)))
        return jnp.maximum(jnp.stack(ys) + b, 0.0)

    x3 = block(x_ref[...], p5_ref[...], l3_ref[...], w3_ref[...], b3_ref[...])
    o3_ref[...] = x3
    o2_ref[...] = block(x3, p1_ref[...], l2_ref[...], w2_ref[...],
                        b2_ref[...])


def _fine_body(x_ref, *refs, relu, has_pm, has_skip, post, banded, fo, nb):
    i = 0
    s_ref = pm_ref = None
    if has_skip:
        s_ref = refs[i]; i += 1
    if has_pm:
        pm_ref = refs[i]; i += 1
    l_ref, w_ref, b_ref = refs[i:i + 3]
    i += 3
    ws_ref = None
    if has_skip and post:
        ws_ref = refs[i]; i += 1
    o_ref = refs[i]

    x3 = x_ref[...]
    fin_raw, vin = x3.shape[1], x3.shape[2]

    if not post:
        xm = x3.reshape(nb * fin_raw, vin)
        if has_pm:
            xm = _dot(xm, pm_ref[...])
        vout = xm.shape[1]
        if has_skip:
            s3 = s_ref[...]
            xm = jnp.concatenate(
                [xm.reshape(nb, fin_raw, vout), s3], axis=1)
            fin = fin_raw + s3.shape[1]
            xm = xm.reshape(nb * fin, vout)
        else:
            fin = fin_raw
        x0 = xm
        x1 = _apply_l(x0, l_ref, banded)
        x2 = 2.0 * _apply_l(x1, l_ref, banded) - x0
        w = w_ref[...]
        ys = []
        for b in range(nb):
            xcb = jnp.concatenate(
                [x0[b * fin:(b + 1) * fin],
                 x1[b * fin:(b + 1) * fin],
                 x2[b * fin:(b + 1) * fin]], axis=0)
            ys.append(_dot(w, xcb))
        y = jnp.stack(ys)
    else:
        w = w_ref[...]
        pieces = []
        for b in range(nb):
            yb = _dot(w, x3[b])
            if has_skip:
                yb = yb + _dot(ws_ref[...], s_ref[b])
            pieces.append(yb)
        y3m = jnp.concatenate(pieces, axis=0)
        if has_pm:
            y3m = _dot(y3m, pm_ref[...])
        vout = y3m.shape[1]
        y3d = y3m.reshape(nb, 3 * fo, vout)
        y0 = y3d[:, :fo, :].reshape(nb * fo, vout)
        y1 = y3d[:, fo:2 * fo, :].reshape(nb * fo, vout)
        y2 = y3d[:, 2 * fo:, :].reshape(nb * fo, vout)
        t = _apply_l(y2, l_ref, banded)
        y = (y0 - y2) + _apply_l(y1 + 2.0 * t, l_ref, banded)
        y = y.reshape(nb, fo, vout)

    y = y + b_ref[...]
    if relu:
        y = jnp.maximum(y, 0.0)
    o_ref[...] = y


def _conv(x, ld, w, b, *, skip=None, pm=None, relu=True, post=False,
          banded=False, coarse=False, nb=1):
    if post:
        (w_main, ws) = w if skip is not None else (w, None)
        fo = w_main.shape[0] // 3
    else:
        w_main, ws = w, None
        fo = w.shape[0]
    vout = ld.shape[0] * ld.shape[2] if banded else ld.shape[-1]

    def const(s):
        return pl.BlockSpec(s, lambda i: tuple(0 for _ in s))

    if coarse:
        bv_out = (pm.shape[1] if pm is not None else x.shape[1])
        inputs = [x]
        in_specs = [const(x.shape)]
        if skip is not None:
            inputs.append(skip); in_specs.append(const(skip.shape))
        if pm is not None:
            inputs.append(pm); in_specs.append(const(pm.shape))
        inputs += [ld, w_main, b.reshape(fo, 1)]
        in_specs += [const(ld.shape), const(w_main.shape), const((fo, 1))]
        if ws is not None:
            inputs.append(ws); in_specs.append(const(ws.shape))
        body = functools.partial(
            _coarse_body, relu=relu, has_pm=pm is not None,
            has_skip=skip is not None, post=post, fo=fo)
        return pl.pallas_call(
            body, grid=(1,), in_specs=in_specs,
            out_specs=const((fo, bv_out)),
            out_shape=jax.ShapeDtypeStruct((fo, bv_out), jnp.float32),
        )(*inputs)

    bsz, fin_raw, vin = x.shape
    inputs = [x]
    in_specs = [pl.BlockSpec((nb, fin_raw, vin), lambda i: (i, 0, 0))]
    if skip is not None:
        fs = skip.shape[1]
        inputs.append(skip)
        in_specs.append(pl.BlockSpec((nb, fs, vout), lambda i: (i, 0, 0)))
    if pm is not None:
        inputs.append(pm)
        in_specs.append(const(pm.shape))
    inputs += [ld, w_main, b.reshape(fo, 1)]
    in_specs += [const(ld.shape), const(w_main.shape), const((fo, 1))]
    if ws is not None:
        inputs.append(ws)
        in_specs.append(const(ws.shape))

    body = functools.partial(
        _fine_body, relu=relu, has_pm=pm is not None,
        has_skip=skip is not None, post=post, banded=banded, fo=fo, nb=nb)
    return pl.pallas_call(
        body, grid=(bsz // nb,), in_specs=in_specs,
        out_specs=pl.BlockSpec((nb, fo, vout), lambda i: (i, 0, 0)),
        out_shape=jax.ShapeDtypeStruct((bsz, fo, vout), jnp.float32),
    )(*inputs)


def _w_pre(params, name):
    w = params[name + '_w']
    k, fin, fo = w.shape
    return w.reshape(k * fin, fo).T, params[name + '_b']


def _w_post(params, name, split=None):
    w = params[name + '_w']
    k, fin, fo = w.shape
    if split is None:
        return w.transpose(0, 2, 1).reshape(k * fo, fin), params[name + '_b']
    wh = w[:, :split, :].transpose(0, 2, 1).reshape(k * fo, split)
    ws = w[:, split:, :].transpose(0, 2, 1).reshape(k * fo, fin - split)
    return (wh, ws), params[name + '_b']


@jax.jit
def kernel(x, params, laps):
    bsz = x.shape[0]

    sizes = [v * v for v in _NODES[:4]]
    win = _BAND_BLK + 2 * _BAND_HALO
    sizes.append((_NODES[4] // _BAND_BLK) * win * _BAND_BLK)
    bases = list(np.cumsum([0] + sizes[:-1]))
    total = int(np.sum(sizes))
    ch = -(-total // (_SC_TECS * 16)) * 16
    pad_total = ch * _SC_TECS

    dst = jnp.concatenate(
        [_dst_dense(laps[i], _NODES[i], int(bases[i])) for i in range(4)]
        + [_dst_band(laps[4], int(bases[4]))])
    vals = jnp.concatenate([laps[i][2] for i in range(5)])
    pad = -(-dst.shape[0] // 16) * 16 - dst.shape[0]
    dst = jnp.pad(dst, (0, pad), constant_values=-1)
    vals = jnp.pad(vals, (0, pad))
    flat = _sc_build_flat(dst, vals, pad_total, ch)

    o = [int(b) for b in bases]
    ld2 = flat[o[1]:o[1] + sizes[1]].reshape(_NODES[1], _NODES[1])
    ld3 = flat[o[2]:o[2] + sizes[2]].reshape(_NODES[2], _NODES[2])
    ld4 = flat[o[3]:o[3] + sizes[3]].reshape(_NODES[3], _NODES[3])
    lw5 = flat[o[4]:o[4] + sizes[4]].reshape(-1, win, _BAND_BLK)
    ld1 = flat[o[0]:o[0] + sizes[0]].reshape(_NODES[0], _NODES[0])
    lk1 = _kron_lift(ld1, bsz)
    lk2 = _kron_lift(ld2, bsz)

    pk32 = jnp.asarray(_PK32)
    uk32 = jnp.asarray(_UK32)
    p2048 = jnp.asarray(_POOL[2048])
    p512 = jnp.asarray(_POOL[512])
    p128 = jnp.asarray(_POOL[128])
    u128 = jnp.asarray(_UNPOOL[128])
    u512 = jnp.asarray(_UNPOOL[512])
    u2048 = jnp.asarray(_UNPOOL[2048])

    xt = jnp.transpose(x, (0, 2, 1))

    def pre(name, ld, h, **kw):
        wt, b = _w_pre(params, name)
        return _conv(h, ld, wt, b, **kw)

    def post(name, ld, h, split=None, **kw):
        wt, b = _w_post(params, name, split)
        return _conv(h, ld, wt, b, post=True, **kw)

    we1, be1 = _w_pre(params, 'conv1_enc_l5')
    we2, be2 = _w_pre(params, 'conv2_enc_l5')
    eins = [xt, lw5, we1, be1.reshape(-1, 1), we2, be2.reshape(-1, 1)]
    especs = [pl.BlockSpec((8, 16, 2048), lambda i: (i, 0, 0))] + [
        pl.BlockSpec(a.shape, lambda i, s=a.shape: tuple(0 for _ in s))
        for a in eins[1:]]
    x5 = pl.pallas_call(
        functools.partial(_enc5_pair_body, nb=8), grid=(bsz // 8,),
        in_specs=especs,
        out_specs=pl.BlockSpec((8, 64, 2048), lambda i: (i, 0, 0)),
        out_shape=jax.ShapeDtypeStruct((bsz, 64, 2048), jnp.float32),
    )(*eins)
    x4 = pre('conv_enc_l4', ld4, x5, pm=p2048, nb=16)
    wl3, bl3 = _w_pre(params, 'conv_enc_l3')
    wl2, bl2 = _w_pre(params, 'conv_enc_l2')
    fins = [x4, ld3, ld2, p512, p128,
            wl3, bl3.reshape(-1, 1), wl2, bl2.reshape(-1, 1)]
    fspecs = [pl.BlockSpec((bsz, 128, 512), lambda i: (i, 0, 0))] + [
        pl.BlockSpec(a.shape, lambda i, s=a.shape: tuple(0 for _ in s))
        for a in fins[1:]]
    x3, x2 = pl.pallas_call(
        functools.partial(_enc32_body, nb=bsz), grid=(1,),
        in_specs=fspecs,
        out_specs=[pl.BlockSpec((bsz, 256, 128), lambda i: (i, 0, 0)),
                   pl.BlockSpec((bsz, 512, 32), lambda i: (i, 0, 0))],
        out_shape=[jax.ShapeDtypeStruct((bsz, 256, 128), jnp.float32),
                   jax.ShapeDtypeStruct((bsz, 512, 32), jnp.float32)],
    )(*fins)
    x2f = jnp.transpose(x2, (1, 0, 2)).reshape(512, bsz * 32)
    wt1, b1 = _w_pre(params, 'conv_enc_l1')
    wt2, b2 = _w_pre(params, 'conv_enc_l0')
    wt3, b3 = _w_pre(params, 'conv1_dec_l1')
    wt4, b4 = _w_pre(params, 'conv2_dec_l1')
    wt5, b5 = _w_post(params, 'conv1_dec_l2')
    (w6h, w6s), b6 = _w_post(params, 'conv2_dec_l2', 256)
    ins = [x2f, lk1, lk2, pk32, uk32,
           wt1, b1.reshape(-1, 1), wt2, b2.reshape(-1, 1),
           wt3, b3.reshape(-1, 1), wt4, b4.reshape(-1, 1),
           wt5, b5.reshape(-1, 1), w6h, w6s, b6.reshape(-1, 1)]

    def cspec(s):
        return pl.BlockSpec(s, lambda i: tuple(0 for _ in s))

    h = pl.pallas_call(
        _coarse_chain_body, grid=(1,),
        in_specs=[cspec(a.shape) for a in ins],
        out_specs=cspec((256, bsz * 32)),
        out_shape=jax.ShapeDtypeStruct((256, bsz * 32), jnp.float32),
    )(*ins)
    h = jnp.transpose(h.reshape(256, bsz, 32), (1, 0, 2))
    def dec_pair(n1, n2, split, h, ld, pm, skip, nb):
        w1, b1 = _w_post(params, n1)
        (w2h, w2s), b2 = _w_post(params, n2, split)
        fo = b2.shape[0]
        vout = pm.shape[1]
        ins = [h, ld, pm, skip, w1, b1.reshape(-1, 1),
               w2h, w2s, b2.reshape(-1, 1)]
        specs = [pl.BlockSpec((nb,) + h.shape[1:], lambda i: (i, 0, 0))]
        specs += [pl.BlockSpec(a.shape,
                               lambda i, s=a.shape: tuple(0 for _ in s))
                  for a in ins[1:3]]
        specs.append(pl.BlockSpec((nb,) + skip.shape[1:],
                                  lambda i: (i, 0, 0)))
        specs += [pl.BlockSpec(a.shape,
                               lambda i, s=a.shape: tuple(0 for _ in s))
                  for a in ins[4:]]
        return pl.pallas_call(
            functools.partial(_dec_pair_body, nb=nb), grid=(bsz // nb,),
            in_specs=specs,
            out_specs=pl.BlockSpec((nb, fo, vout), lambda i: (i, 0, 0)),
            out_shape=jax.ShapeDtypeStruct((bsz, fo, vout), jnp.float32),
        )(*ins)

    h = dec_pair('conv1_dec_l3', 'conv2_dec_l3', 128, h, ld3, u128, x3, 32)
    h = dec_pair('conv1_dec_l4', 'conv2_dec_l4', 64, h, ld4, u512, x4, 16)
    wd1, bd1 = _w_post(params, 'conv1_dec_l5')
    wd2, bd2 = _w_post(params, 'conv2_dec_l5')
    dins = [h, lw5, u2048, wd1, bd1.reshape(-1, 1), wd2, bd2.reshape(-1, 1)]
    dspecs = [pl.BlockSpec((8, 64, 512), lambda i: (i, 0, 0))] + [
        pl.BlockSpec(a.shape, lambda i, s=a.shape: tuple(0 for _ in s))
        for a in dins[1:]]
    h = pl.pallas_call(
        functools.partial(_dec5_pair_body, nb=8), grid=(bsz // 8,),
        in_specs=dspecs,
        out_specs=pl.BlockSpec((8, 16, 2048), lambda i: (i, 0, 0)),
        out_shape=jax.ShapeDtypeStruct((bsz, 16, 2048), jnp.float32),
    )(*dins)

    return jnp.transpose(h, (0, 2, 1))


# speedup vs baseline: 1.0234x; 1.0234x over previous
import functools

import jax
import jax.numpy as jnp
import numpy as np
from jax import lax
from jax.experimental import pallas as pl
from jax.experimental.pallas import tpu as pltpu
from jax.experimental.pallas import tpu_sc as plsc

_NODES = [8, 32, 128, 512, 2048]
_BAND_BLK = 256
_BAND_HALO = 128
_BSZ = 32


def _pool_matrix(v):
    h = int(round((v / 2) ** 0.5))
    w = 2 * h
    p = np.zeros((v, v // 4), np.float32)
    for h2 in range(h // 2):
        for w2 in range(w // 2):
            col = h2 * (w // 2) + w2
            for dh in (0, 1):
                for dw in (0, 1):
                    p[(2 * h2 + dh) * w + (2 * w2 + dw), col] = 0.25
    return p


_POOL = {v: _pool_matrix(v) for v in _NODES[1:]}
_UNPOOL = {v: (4.0 * _POOL[v].T).copy() for v in _NODES[1:]}
_EYE = np.eye(_BSZ, dtype=np.float32)
_PK32 = np.kron(_EYE, _POOL[32])
_UK32 = np.kron(_EYE, _UNPOOL[32])


_SC_TECS = 32


def _sc_build_flat(dst, vals, pad_total, ch):
    tot = dst.shape[0]
    mesh = plsc.VectorSubcoreMesh(core_axis_name="c", subcore_axis_name="s")

    def body(dst_hbm, vals_hbm, out_hbm, dst_v, vals_v, chunk_v):
        wid = lax.axis_index("s") * 2 + lax.axis_index("c")
        lo = wid * ch
        pltpu.sync_copy(dst_hbm, dst_v)
        pltpu.sync_copy(vals_hbm, vals_v)
        zv = jnp.zeros((16,), jnp.float32)

        def zbody(i, carry):
            chunk_v[pl.ds(i * 16, 16)] = zv
            return carry

        lax.fori_loop(0, ch // 16, zbody, 0)

        def sbody(i, carry):
            d = dst_v[pl.ds(i * 16, 16)]
            v = vals_v[pl.ds(i * 16, 16)]
            dl = d - lo
            m = (d >= lo) & (dl < ch)
            plsc.store_scatter(chunk_v, [dl], v, mask=m)
            return carry

        lax.fori_loop(0, tot // 16, sbody, 0)
        pltpu.sync_copy(chunk_v, out_hbm.at[pl.ds(lo, ch)])

    return pl.kernel(
        body,
        out_type=jax.ShapeDtypeStruct((pad_total,), jnp.float32),
        mesh=mesh,
        compiler_params=pltpu.CompilerParams(needs_layout_passes=False),
        scratch_types=[
            pltpu.VMEM((tot,), jnp.int32),
            pltpu.VMEM((tot,), jnp.float32),
            pltpu.VMEM((ch,), jnp.float32),
        ],
    )(dst, vals)


def _dst_dense(lap, v, base):
    rows, cols, _ = lap
    return base + rows * v + cols


def _dst_band(lap, base):
    rows, cols, _ = lap
    j = cols // _BAND_BLK
    rloc = rows - j * _BAND_BLK + _BAND_HALO
    win = _BAND_BLK + 2 * _BAND_HALO
    return base + (j * win + rloc) * _BAND_BLK + cols % _BAND_BLK


def _kron_lift(d, bsz):
    v = d.shape[0]
    eye = jnp.asarray(np.eye(bsz, dtype=np.float32))
    return (eye[:, None, :, None] * d[None, :, None, :]).reshape(
        bsz * v, bsz * v)


def _dot(a, b):
    return jnp.dot(a, b, preferred_element_type=jnp.float32)


def _apply_l(z, l_ref, banded):
    if not banded:
        return _dot(z, l_ref[...])
    nblk = l_ref.shape[0]
    zp = jnp.pad(z, ((0, 0), (_BAND_HALO, _BAND_HALO)))
    win = _BAND_BLK + 2 * _BAND_HALO
    outs = []
    for j in range(nblk):
        outs.append(_dot(zp[:, j * _BAND_BLK:j * _BAND_BLK + win], l_ref[j]))
    return jnp.concatenate(outs, axis=1)


def _coarse_body(x_ref, *refs, relu, has_pm, has_skip, post, fo):
    i = 0
    s_ref = pm_ref = None
    if has_skip:
        s_ref = refs[i]; i += 1
    if has_pm:
        pm_ref = refs[i]; i += 1
    l_ref, w_ref, b_ref = refs[i:i + 3]
    i += 3
    ws_ref = None
    if has_skip and post:
        ws_ref = refs[i]; i += 1
    o_ref = refs[i]

    x = x_ref[...]
    if not post:
        if has_pm:
            x = _dot(x, pm_ref[...])
        if has_skip:
            x = jnp.concatenate([x, s_ref[...]], axis=0)
        x0 = x
        x1 = _dot(x0, l_ref[...])
        x2 = 2.0 * _dot(x1, l_ref[...]) - x0
        y = _dot(w_ref[...], jnp.concatenate([x0, x1, x2], axis=0))
    else:
        y3 = _dot(w_ref[...], x)
        if has_skip:
            y3 = y3 + _dot(ws_ref[...], s_ref[...])
        if has_pm:
            y3 = _dot(y3, pm_ref[...])
        y0, y1, y2 = y3[:fo], y3[fo:2 * fo], y3[2 * fo:]
        t = _dot(y2, l_ref[...])
        y = (y0 - y2) + _dot(y1 + 2.0 * t, l_ref[...])

    y = y + b_ref[...]
    if relu:
        y = jnp.maximum(y, 0.0)
    o_ref[...] = y


def _cheb_pre(x, l, w, b, relu):
    x1 = _dot(x, l)
    x2 = 2.0 * _dot(x1, l) - x
    y = _dot(w, jnp.concatenate([x, x1, x2], axis=0)) + b
    return jnp.maximum(y, 0.0) if relu else y


def _cheb_post(y3, l, b, fo, relu):
    y0, y1, y2 = y3[:fo], y3[fo:2 * fo], y3[2 * fo:]
    t = _dot(y2, l)
    y = (y0 - y2) + _dot(y1 + 2.0 * t, l) + b
    return jnp.maximum(y, 0.0) if relu else y


def _coarse_chain_body(x2f_ref, lk1_ref, lk2_ref, pk_ref, uk_ref,
                       w1_ref, b1_ref, w2_ref, b2_ref, w3_ref, b3_ref,
                       w4_ref, b4_ref, w5_ref, b5_ref,
                       w6h_ref, w6s_ref, b6_ref, o_ref):
    x2f = x2f_ref[...]
    lk1 = lk1_ref[...]
    lk2 = lk2_ref[...]
    x1f = _cheb_pre(_dot(x2f, pk_ref[...]), lk1, w1_ref[...], b1_ref[...],
                    True)
    x0f = _cheb_pre(x1f, lk1, w2_ref[...], b2_ref[...], False)
    h = _cheb_pre(x0f, lk1, w3_ref[...], b3_ref[...], True)
    h = _cheb_pre(jnp.concatenate([h, x1f], axis=0), lk1, w4_ref[...],
                  b4_ref[...], True)
    y3 = _dot(_dot(w5_ref[...], h), uk_ref[...])
    fo = b5_ref.shape[0]
    h = _cheb_post(y3, lk2, b5_ref[...], fo, True)
    y3 = _dot(w6h_ref[...], h) + _dot(w6s_ref[...], x2f)
    o_ref[...] = _cheb_post(y3, lk2, b6_ref[...], fo, True)


def _enc5_pair_body(x_ref, l_ref, w1_ref, b1_ref, w2_ref, b2_ref, o_ref, *,
                    nb):
    def cheb(x3, w, b):
        f, v = x3.shape[1], x3.shape[2]
        xm = x3.reshape(nb * f, v)
        x1 = _apply_l(xm, l_ref, True)
        x2 = 2.0 * _apply_l(x1, l_ref, True) - xm
        ys = []
        for bi in range(nb):
            s = slice(bi * f, (bi + 1) * f)
            ys.append(_dot(w, jnp.concatenate([xm[s], x1[s], x2[s]], axis=0)))
        return jnp.stack(ys) + b

    y = jnp.maximum(cheb(x_ref[...], w1_ref[...], b1_ref[...]), 0.0)
    o_ref[...] = jnp.maximum(cheb(y, w2_ref[...], b2_ref[...]), 0.0)


def _dec5_pair_body(x_ref, l_ref, pm_ref, w1_ref, b1_ref, w2_ref, b2_ref,
                    o_ref, *, nb):
    def cheb_post(x3, w, b, pm):
        fo3 = w.shape[0]
        fo = fo3 // 3
        y3m = jnp.concatenate([_dot(w, x3[bi]) for bi in range(nb)], axis=0)
        if pm is not None:
            y3m = _dot(y3m, pm)
        v = y3m.shape[1]
        y3d = y3m.reshape(nb, fo3, v)
        y0 = y3d[:, :fo, :].reshape(nb * fo, v)
        y1 = y3d[:, fo:2 * fo, :].reshape(nb * fo, v)
        y2 = y3d[:, 2 * fo:, :].reshape(nb * fo, v)
        t = _apply_l(y2, l_ref, True)
        y = (y0 - y2) + _apply_l(y1 + 2.0 * t, l_ref, True)
        return y.reshape(nb, fo, v) + b

    y = cheb_post(x_ref[...], w1_ref[...], b1_ref[...], pm_ref[...])
    o_ref[...] = cheb_post(y, w2_ref[...], b2_ref[...], None)


def _dec_pair_body(x_ref, l_ref, pm_ref, s_ref, w1_ref, b1_ref, w2h_ref,
                   w2s_ref, b2_ref, o_ref, *, nb):
    def cheb_post(x3, w, b, pm, skip3, ws):
        fo3 = w.shape[0]
        fo = fo3 // 3
        parts = []
        for bi in range(nb):
            yb = _dot(w, x3[bi])
            if skip3 is not None:
                yb = yb + _dot(ws, skip3[bi])
            parts.append(yb)
        y3m = jnp.concatenate(parts, axis=0)
        if pm is not None:
            y3m = _dot(y3m, pm)
        v = y3m.shape[1]
        y3d = y3m.reshape(nb, fo3, v)
        y0 = y3d[:, :fo, :].reshape(nb * fo, v)
        y1 = y3d[:, fo:2 * fo, :].reshape(nb * fo, v)
        y2 = y3d[:, 2 * fo:, :].reshape(nb * fo, v)
        t = _dot(y2, l_ref[...])
        y = (y0 - y2) + _dot(y1 + 2.0 * t, l_ref[...])
        return jnp.maximum(y.reshape(nb, fo, v) + b, 0.0)

    y = cheb_post(x_ref[...], w1_ref[...], b1_ref[...], pm_ref[...],
                  None, None)
    o_ref[...] = cheb_post(y, w2h_ref[...], b2_ref[...], None,
                           s_ref[...], w2s_ref[...])


def _fine_body(x_ref, *refs, relu, has_pm, has_skip, post, banded, fo, nb):
    i = 0
    s_ref = pm_ref = None
    if has_skip:
        s_ref = refs[i]; i += 1
    if has_pm:
        pm_ref = refs[i]; i += 1
    l_ref, w_ref, b_ref = refs[i:i + 3]
    i += 3
    ws_ref = None
    if has_skip and post:
        ws_ref = refs[i]; i += 1
    o_ref = refs[i]

    x3 = x_ref[...]
    fin_raw, vin = x3.shape[1], x3.shape[2]

    if not post:
        xm = x3.reshape(nb * fin_raw, vin)
        if has_pm:
            xm = _dot(xm, pm_ref[...])
        vout = xm.shape[1]
        if has_skip:
            s3 = s_ref[...]
            xm = jnp.concatenate(
                [xm.reshape(nb, fin_raw, vout), s3], axis=1)
            fin = fin_raw + s3.shape[1]
            xm = xm.reshape(nb * fin, vout)
        else:
            fin = fin_raw
        x0 = xm
        x1 = _apply_l(x0, l_ref, banded)
        x2 = 2.0 * _apply_l(x1, l_ref, banded) - x0
        w = w_ref[...]
        ys = []
        for b in range(nb):
            xcb = jnp.concatenate(
                [x0[b * fin:(b + 1) * fin],
                 x1[b * fin:(b + 1) * fin],
                 x2[b * fin:(b + 1) * fin]], axis=0)
            ys.append(_dot(w, xcb))
        y = jnp.stack(ys)
    else:
        w = w_ref[...]
        pieces = []
        for b in range(nb):
            yb = _dot(w, x3[b])
            if has_skip:
                yb = yb + _dot(ws_ref[...], s_ref[b])
            pieces.append(yb)
        y3m = jnp.concatenate(pieces, axis=0)
        if has_pm:
            y3m = _dot(y3m, pm_ref[...])
        vout = y3m.shape[1]
        y3d = y3m.reshape(nb, 3 * fo, vout)
        y0 = y3d[:, :fo, :].reshape(nb * fo, vout)
        y1 = y3d[:, fo:2 * fo, :].reshape(nb * fo, vout)
        y2 = y3d[:, 2 * fo:, :].reshape(nb * fo, vout)
        t = _apply_l(y2, l_ref, banded)
        y = (y0 - y2) + _apply_l(y1 + 2.0 * t, l_ref, banded)
        y = y.reshape(nb, fo, vout)

    y = y + b_ref[...]
    if relu:
        y = jnp.maximum(y, 0.0)
    o_ref[...] = y


def _conv(x, ld, w, b, *, skip=None, pm=None, relu=True, post=False,
          banded=False, coarse=False, nb=1):
    if post:
        (w_main, ws) = w if skip is not None else (w, None)
        fo = w_main.shape[0] // 3
    else:
        w_main, ws = w, None
        fo = w.shape[0]
    vout = ld.shape[0] * ld.shape[2] if banded else ld.shape[-1]

    def const(s):
        return pl.BlockSpec(s, lambda i: tuple(0 for _ in s))

    if coarse:
        bv_out = (pm.shape[1] if pm is not None else x.shape[1])
        inputs = [x]
        in_specs = [const(x.shape)]
        if skip is not None:
            inputs.append(skip); in_specs.append(const(skip.shape))
        if pm is not None:
            inputs.append(pm); in_specs.append(const(pm.shape))
        inputs += [ld, w_main, b.reshape(fo, 1)]
        in_specs += [const(ld.shape), const(w_main.shape), const((fo, 1))]
        if ws is not None:
            inputs.append(ws); in_specs.append(const(ws.shape))
        body = functools.partial(
            _coarse_body, relu=relu, has_pm=pm is not None,
            has_skip=skip is not None, post=post, fo=fo)
        return pl.pallas_call(
            body, grid=(1,), in_specs=in_specs,
            out_specs=const((fo, bv_out)),
            out_shape=jax.ShapeDtypeStruct((fo, bv_out), jnp.float32),
        )(*inputs)

    bsz, fin_raw, vin = x.shape
    inputs = [x]
    in_specs = [pl.BlockSpec((nb, fin_raw, vin), lambda i: (i, 0, 0))]
    if skip is not None:
        fs = skip.shape[1]
        inputs.append(skip)
        in_specs.append(pl.BlockSpec((nb, fs, vout), lambda i: (i, 0, 0)))
    if pm is not None:
        inputs.append(pm)
        in_specs.append(const(pm.shape))
    inputs += [ld, w_main, b.reshape(fo, 1)]
    in_specs += [const(ld.shape), const(w_main.shape), const((fo, 1))]
    if ws is not None:
        inputs.append(ws)
        in_specs.append(const(ws.shape))

    body = functools.partial(
        _fine_body, relu=relu, has_pm=pm is not None,
        has_skip=skip is not None, post=post, banded=banded, fo=fo, nb=nb)
    return pl.pallas_call(
        body, grid=(bsz // nb,), in_specs=in_specs,
        out_specs=pl.BlockSpec((nb, fo, vout), lambda i: (i, 0, 0)),
        out_shape=jax.ShapeDtypeStruct((bsz, fo, vout), jnp.float32),
    )(*inputs)


def _w_pre(params, name):
    w = params[name + '_w']
    k, fin, fo = w.shape
    return w.reshape(k * fin, fo).T, params[name + '_b']


def _w_post(params, name, split=None):
    w = params[name + '_w']
    k, fin, fo = w.shape
    if split is None:
        return w.transpose(0, 2, 1).reshape(k * fo, fin), params[name + '_b']
    wh = w[:, :split, :].transpose(0, 2, 1).reshape(k * fo, split)
    ws = w[:, split:, :].transpose(0, 2, 1).reshape(k * fo, fin - split)
    return (wh, ws), params[name + '_b']


@jax.jit
def kernel(x, params, laps):
    bsz = x.shape[0]

    sizes = [v * v for v in _NODES[:4]]
    win = _BAND_BLK + 2 * _BAND_HALO
    sizes.append((_NODES[4] // _BAND_BLK) * win * _BAND_BLK)
    bases = list(np.cumsum([0] + sizes[:-1]))
    total = int(np.sum(sizes))
    ch = -(-total // (_SC_TECS * 16)) * 16
    pad_total = ch * _SC_TECS

    dst = jnp.concatenate(
        [_dst_dense(laps[i], _NODES[i], int(bases[i])) for i in range(4)]
        + [_dst_band(laps[4], int(bases[4]))])
    vals = jnp.concatenate([laps[i][2] for i in range(5)])
    pad = -(-dst.shape[0] // 16) * 16 - dst.shape[0]
    dst = jnp.pad(dst, (0, pad), constant_values=-1)
    vals = jnp.pad(vals, (0, pad))
    flat = _sc_build_flat(dst, vals, pad_total, ch)

    o = [int(b) for b in bases]
    ld2 = flat[o[1]:o[1] + sizes[1]].reshape(_NODES[1], _NODES[1])
    ld3 = flat[o[2]:o[2] + sizes[2]].reshape(_NODES[2], _NODES[2])
    ld4 = flat[o[3]:o[3] + sizes[3]].reshape(_NODES[3], _NODES[3])
    lw5 = flat[o[4]:o[4] + sizes[4]].reshape(-1, win, _BAND_BLK)
    ld1 = flat[o[0]:o[0] + sizes[0]].reshape(_NODES[0], _NODES[0])
    lk1 = _kron_lift(ld1, bsz)
    lk2 = _kron_lift(ld2, bsz)

    pk32 = jnp.asarray(_PK32)
    uk32 = jnp.asarray(_UK32)
    p2048 = jnp.asarray(_POOL[2048])
    p512 = jnp.asarray(_POOL[512])
    p128 = jnp.asarray(_POOL[128])
    u128 = jnp.asarray(_UNPOOL[128])
    u512 = jnp.asarray(_UNPOOL[512])
    u2048 = jnp.asarray(_UNPOOL[2048])

    xt = jnp.transpose(x, (0, 2, 1))

    def pre(name, ld, h, **kw):
        wt, b = _w_pre(params, name)
        return _conv(h, ld, wt, b, **kw)

    def post(name, ld, h, split=None, **kw):
        wt, b = _w_post(params, name, split)
        return _conv(h, ld, wt, b, post=True, **kw)

    we1, be1 = _w_pre(params, 'conv1_enc_l5')
    we2, be2 = _w_pre(params, 'conv2_enc_l5')
    eins = [xt, lw5, we1, be1.reshape(-1, 1), we2, be2.reshape(-1, 1)]
    especs = [pl.BlockSpec((8, 16, 2048), lambda i: (i, 0, 0))] + [
        pl.BlockSpec(a.shape, lambda i, s=a.shape: tuple(0 for _ in s))
        for a in eins[1:]]
    x5 = pl.pallas_call(
        functools.partial(_enc5_pair_body, nb=8), grid=(bsz // 8,),
        in_specs=especs,
        out_specs=pl.BlockSpec((8, 64, 2048), lambda i: (i, 0, 0)),
        out_shape=jax.ShapeDtypeStruct((bsz, 64, 2048), jnp.float32),
    )(*eins)
    x4 = pre('conv_enc_l4', ld4, x5, pm=p2048, nb=16)
    x3 = pre('conv_enc_l3', ld3, x4, pm=p512, nb=32)
    x2 = pre('conv_enc_l2', ld2, x3, pm=p128, nb=32)
    x2f = jnp.transpose(x2, (1, 0, 2)).reshape(512, bsz * 32)
    wt1, b1 = _w_pre(params, 'conv_enc_l1')
    wt2, b2 = _w_pre(params, 'conv_enc_l0')
    wt3, b3 = _w_pre(params, 'conv1_dec_l1')
    wt4, b4 = _w_pre(params, 'conv2_dec_l1')
    wt5, b5 = _w_post(params, 'conv1_dec_l2')
    (w6h, w6s), b6 = _w_post(params, 'conv2_dec_l2', 256)
    ins = [x2f, lk1, lk2, pk32, uk32,
           wt1, b1.reshape(-1, 1), wt2, b2.reshape(-1, 1),
           wt3, b3.reshape(-1, 1), wt4, b4.reshape(-1, 1),
           wt5, b5.reshape(-1, 1), w6h, w6s, b6.reshape(-1, 1)]

    def cspec(s):
        return pl.BlockSpec(s, lambda i: tuple(0 for _ in s))

    h = pl.pallas_call(
        _coarse_chain_body, grid=(1,),
        in_specs=[cspec(a.shape) for a in ins],
        out_specs=cspec((256, bsz * 32)),
        out_shape=jax.ShapeDtypeStruct((256, bsz * 32), jnp.float32),
    )(*ins)
    h = jnp.transpose(h.reshape(256, bsz, 32), (1, 0, 2))
    def dec_pair(n1, n2, split, h, ld, pm, skip, nb):
        w1, b1 = _w_post(params, n1)
        (w2h, w2s), b2 = _w_post(params, n2, split)
        fo = b2.shape[0]
        vout = pm.shape[1]
        ins = [h, ld, pm, skip, w1, b1.reshape(-1, 1),
               w2h, w2s, b2.reshape(-1, 1)]
        specs = [pl.BlockSpec((nb,) + h.shape[1:], lambda i: (i, 0, 0))]
        specs += [pl.BlockSpec(a.shape,
                               lambda i, s=a.shape: tuple(0 for _ in s))
                  for a in ins[1:3]]
        specs.append(pl.BlockSpec((nb,) + skip.shape[1:],
                                  lambda i: (i, 0, 0)))
        specs += [pl.BlockSpec(a.shape,
                               lambda i, s=a.shape: tuple(0 for _ in s))
                  for a in ins[4:]]
        return pl.pallas_call(
            functools.partial(_dec_pair_body, nb=nb), grid=(bsz // nb,),
            in_specs=specs,
            out_specs=pl.BlockSpec((nb, fo, vout), lambda i: (i, 0, 0)),
            out_shape=jax.ShapeDtypeStruct((bsz, fo, vout), jnp.float32),
        )(*ins)

    h = dec_pair('conv1_dec_l3', 'conv2_dec_l3', 128, h, ld3, u128, x3, 32)
    h = dec_pair('conv1_dec_l4', 'conv2_dec_l4', 64, h, ld4, u512, x4, 16)
    wd1, bd1 = _w_post(params, 'conv1_dec_l5')
    wd2, bd2 = _w_post(params, 'conv2_dec_l5')
    dins = [h, lw5, u2048, wd1, bd1.reshape(-1, 1), wd2, bd2.reshape(-1, 1)]
    dspecs = [pl.BlockSpec((8, 64, 512), lambda i: (i, 0, 0))] + [
        pl.BlockSpec(a.shape, lambda i, s=a.shape: tuple(0 for _ in s))
        for a in dins[1:]]
    h = pl.pallas_call(
        functools.partial(_dec5_pair_body, nb=8), grid=(bsz // 8,),
        in_specs=dspecs,
        out_specs=pl.BlockSpec((8, 16, 2048), lambda i: (i, 0, 0)),
        out_shape=jax.ShapeDtypeStruct((bsz, 16, 2048), jnp.float32),
    )(*dins)

    return jnp.transpose(h, (0, 2, 1))
